# baseline (speedup 1.0000x reference)
import functools

import jax
import jax.numpy as jnp
from jax.experimental import pallas as pl
from jax.experimental.pallas import tpu as pltpu


_F = 16
_HID = 32
_D = 3
_ND = (2 * _D + 1) ** 2
_PAD = 128
_DEC_CH = 80


def _leaky(v):
    return jnp.where(v > 0, v, 0.1 * v)


def _shifts(dmax, W):
    return [dy * W + dx
            for dy in range(-dmax, dmax + 1)
            for dx in range(-dmax, dmax + 1)]


def _pad_lanes(x):
    z = jnp.zeros((x.shape[0], _PAD), x.dtype)
    return jnp.concatenate([z, x, z], axis=-1)


def _conv3x3_bf16(xb, w_ref, b_ref, cmask_ref, W):
    cin, L = xb.shape
    xp = _pad_lanes(xb)
    cols = []
    k = 0
    for dy in (-1, 0, 1):
        for dx in (-1, 0, 1):
            s = dy * W + dx
            sh = xp[:, _PAD + s:_PAD + s + L]
            m = cmask_ref[k]
            if cin <= _F:
                mc = m[:cin]
            else:
                mc = jnp.concatenate([m] * (cin // _F), axis=0)
            cols.append(mc * sh)
            k += 1
    patches = jnp.concatenate(cols, axis=0)
    return jnp.dot(w_ref[...], patches,
                   preferred_element_type=jnp.float32) + b_ref[...]


def _fused_kernel(x1_ref, x2_ref,
                  cmask_ref,
                  kmask_ref,
                  sel_ref,
                  w1_ref, b1_ref, w2_ref, b2_ref,
                  w3_ref, b3_ref, w4_ref, b4_ref,
                  o_ref,
                  *, H, W, BP):
    NP = H * W
    L = BP * NP

    def extract(x):
        h1 = _leaky(_conv3x3_bf16(x, w1_ref, b1_ref, cmask_ref, W)
                    .astype(jnp.bfloat16))
        h2 = _leaky(_conv3x3_bf16(h1, w2_ref, b2_ref, cmask_ref, W)
                    .astype(jnp.bfloat16))
        return h2

    f1b = extract(x1_ref[0])
    f2b = extract(x2_ref[0])

    f2p = _pad_lanes(f2b)
    prods = [f1b * f2p[:, _PAD + s:_PAD + s + L] for s in _shifts(_D, W)]
    pstack = jnp.concatenate(prods, axis=0)
    corr_raw = jnp.dot(sel_ref[...], pstack,
                       preferred_element_type=jnp.float32)
    corr = _leaky(kmask_ref[...] * corr_raw.astype(jnp.bfloat16))

    zpad = jnp.zeros((_DEC_CH - _F - _ND, L), jnp.bfloat16)
    dec_in = jnp.concatenate([f1b, corr, zpad], axis=0)
    h3 = _leaky(_conv3x3_bf16(dec_in, w3_ref, b3_ref, cmask_ref, W)
                .astype(jnp.bfloat16))
    flow = _conv3x3_bf16(h3, w4_ref, b4_ref, cmask_ref, W)

    for bl in range(BP):
        o_ref[bl] = flow[:, bl * NP:(bl + 1) * NP]


def _im2col_w(w):
    cout = w.shape[3]
    return jnp.transpose(w, (3, 0, 1, 2)).reshape(cout, -1).astype(jnp.bfloat16)


def _masks(L, NP, H, W, dmax, dtype):
    p = jnp.arange(L, dtype=jnp.int32) % NP
    hh = p // W
    ww = p - hh * W
    rows = []
    for dy in range(-dmax, dmax + 1):
        for dx in range(-dmax, dmax + 1):
            m = ((hh + dy >= 0) & (hh + dy < H)
                 & (ww + dx >= 0) & (ww + dx < W))
            rows.append(m)
    return jnp.stack(rows).astype(dtype)


def kernel(data, feat1_w, feat1_b, feat2_w, feat2_b,
           dec1_w, dec1_b, dec2_w, dec2_b):
    B, C, H, W = data.shape
    assert C == 6
    NP = H * W
    BP = next(bp for bp in (2, 1) if B % bp == 0)
    L = BP * NP

    x = (data.astype(jnp.bfloat16)
         .reshape(B, 2, 3, NP).transpose(1, 2, 0, 3).reshape(2, 3, B * NP))

    w1 = _im2col_w(feat1_w)
    w2 = _im2col_w(feat2_w)
    w4 = _im2col_w(dec2_w)
    w3o = _im2col_w(dec1_w).reshape(_HID, 9, _ND + _F)
    w3 = jnp.concatenate(
        [w3o[:, :, _ND:], w3o[:, :, :_ND],
         jnp.zeros((_HID, 9, _DEC_CH - _ND - _F), jnp.bfloat16)],
        axis=2).reshape(_HID, 9 * _DEC_CH)

    cmask = jnp.broadcast_to(
        _masks(L, NP, H, W, 1, jnp.bfloat16)[:, None, :],
        (9, _F, L)).copy()
    kmask = (_masks(L, NP, H, W, _D, jnp.float32) / _F
             ).astype(jnp.bfloat16)
    sel = jnp.kron(jnp.eye(_ND, dtype=jnp.bfloat16),
                   jnp.ones((1, _F), jnp.bfloat16))

    def rep(arr):
        nd = arr.ndim
        return pl.BlockSpec(arr.shape, lambda i, _nd=nd: (0,) * _nd)

    kernel_fn = functools.partial(_fused_kernel, H=H, W=W, BP=BP)

    flow = pl.pallas_call(
        kernel_fn,
        out_shape=jax.ShapeDtypeStruct((B, 2, NP), jnp.float32),
        grid=(B // BP,),
        in_specs=[
            pl.BlockSpec((1, 3, L), lambda i: (0, 0, i)),
            pl.BlockSpec((1, 3, L), lambda i: (1, 0, i)),
            rep(cmask), rep(kmask), rep(sel),
            rep(w1), rep(feat1_b), rep(w2), rep(feat2_b),
            rep(w3), rep(dec1_b), rep(w4), rep(dec2_b),
        ],
        out_specs=pl.BlockSpec((BP, 2, NP), lambda i: (i, 0, 0)),
        compiler_params=pltpu.CompilerParams(
            dimension_semantics=("parallel",),
        ),
    )(x, x, cmask, kmask, sel,
      w1, feat1_b, w2, feat2_b, w3, dec1_b, w4, dec2_b)

    return flow.reshape(B, 2, H, W)


# speedup vs baseline: 1.0604x; 1.0604x over previous
import functools

import jax
import jax.numpy as jnp
from jax.experimental import pallas as pl
from jax.experimental.pallas import tpu as pltpu


_F = 16
_HID = 32
_D = 3
_ND = (2 * _D + 1) ** 2
_PAD = 128
_DEC_CH = 80


def _leaky(v):
    return jnp.where(v > 0, v, 0.1 * v)


def _shifts(dmax, W):
    return [dy * W + dx
            for dy in range(-dmax, dmax + 1)
            for dx in range(-dmax, dmax + 1)]


def _pad_lanes(x):
    z = jnp.zeros((x.shape[0], _PAD), x.dtype)
    return jnp.concatenate([z, x, z], axis=-1)


def _conv3x3_bf16(xb, w_ref, b_ref, cmask_ref, W):
    cin, L = xb.shape
    xp = _pad_lanes(xb)
    cols = []
    k = 0
    for dy in (-1, 0, 1):
        for dx in (-1, 0, 1):
            s = dy * W + dx
            sh = xp[:, _PAD + s:_PAD + s + L]
            m = cmask_ref[k]
            if cin <= _F:
                mc = m[:cin]
            else:
                mc = jnp.concatenate([m] * (cin // _F), axis=0)
            cols.append(mc * sh)
            k += 1
    patches = jnp.concatenate(cols, axis=0)
    return jnp.dot(w_ref[...], patches,
                   preferred_element_type=jnp.float32) + b_ref[...]


def _fused_kernel(x1_ref, x2_ref,
                  cmask_ref,
                  kmask_ref,
                  sel_ref,
                  w1_ref, b1_ref, w2_ref, b2_ref,
                  w3_ref, b3_ref, w4_ref, b4_ref,
                  o_ref,
                  *, H, W, BP):
    NP = H * W
    L = BP * NP

    def extract(x):
        h1 = _leaky(_conv3x3_bf16(x, w1_ref, b1_ref, cmask_ref, W)
                    .astype(jnp.bfloat16))
        h2 = _leaky(_conv3x3_bf16(h1, w2_ref, b2_ref, cmask_ref, W)
                    .astype(jnp.bfloat16))
        return h2

    f1b = extract(x1_ref[0])
    f2b = extract(x2_ref[0])

    f2p = _pad_lanes(f2b)
    prods = [f1b * f2p[:, _PAD + s:_PAD + s + L] for s in _shifts(_D, W)]
    pstack = jnp.concatenate(prods, axis=0)
    corr_raw = jnp.dot(sel_ref[...], pstack,
                       preferred_element_type=jnp.float32)
    corr = _leaky(kmask_ref[...] * corr_raw.astype(jnp.bfloat16))

    zpad = jnp.zeros((_DEC_CH - _F - _ND, L), jnp.bfloat16)
    dec_in = jnp.concatenate([f1b, corr, zpad], axis=0)
    h3 = _leaky(_conv3x3_bf16(dec_in, w3_ref, b3_ref, cmask_ref, W)
                .astype(jnp.bfloat16))
    flow = _conv3x3_bf16(h3, w4_ref, b4_ref, cmask_ref, W)

    for bl in range(BP):
        o_ref[bl] = flow[:, bl * NP:(bl + 1) * NP]


def _im2col_w(w):
    cout = w.shape[3]
    return jnp.transpose(w, (3, 0, 1, 2)).reshape(cout, -1).astype(jnp.bfloat16)


def _masks(L, NP, H, W, dmax, dtype):
    p = jnp.arange(L, dtype=jnp.int32) % NP
    hh = p // W
    ww = p - hh * W
    rows = []
    for dy in range(-dmax, dmax + 1):
        for dx in range(-dmax, dmax + 1):
            m = ((hh + dy >= 0) & (hh + dy < H)
                 & (ww + dx >= 0) & (ww + dx < W))
            rows.append(m)
    return jnp.stack(rows).astype(dtype)


def kernel(data, feat1_w, feat1_b, feat2_w, feat2_b,
           dec1_w, dec1_b, dec2_w, dec2_b):
    B, C, H, W = data.shape
    assert C == 6
    NP = H * W
    BP = next(bp for bp in (8, 4, 2, 1) if B % bp == 0)
    L = BP * NP

    x = (data.astype(jnp.bfloat16)
         .reshape(B, 2, 3, NP).transpose(1, 2, 0, 3).reshape(2, 3, B * NP))

    w1 = _im2col_w(feat1_w)
    w2 = _im2col_w(feat2_w)
    w4 = _im2col_w(dec2_w)
    w3o = _im2col_w(dec1_w).reshape(_HID, 9, _ND + _F)
    w3 = jnp.concatenate(
        [w3o[:, :, _ND:], w3o[:, :, :_ND],
         jnp.zeros((_HID, 9, _DEC_CH - _ND - _F), jnp.bfloat16)],
        axis=2).reshape(_HID, 9 * _DEC_CH)

    cmask = jnp.broadcast_to(
        _masks(L, NP, H, W, 1, jnp.bfloat16)[:, None, :],
        (9, _F, L)).copy()
    kmask = (_masks(L, NP, H, W, _D, jnp.float32) / _F
             ).astype(jnp.bfloat16)
    sel = jnp.kron(jnp.eye(_ND, dtype=jnp.bfloat16),
                   jnp.ones((1, _F), jnp.bfloat16))

    def rep(arr):
        nd = arr.ndim
        return pl.BlockSpec(arr.shape, lambda i, _nd=nd: (0,) * _nd)

    kernel_fn = functools.partial(_fused_kernel, H=H, W=W, BP=BP)

    flow = pl.pallas_call(
        kernel_fn,
        out_shape=jax.ShapeDtypeStruct((B, 2, NP), jnp.float32),
        grid=(B // BP,),
        in_specs=[
            pl.BlockSpec((1, 3, L), lambda i: (0, 0, i)),
            pl.BlockSpec((1, 3, L), lambda i: (1, 0, i)),
            rep(cmask), rep(kmask), rep(sel),
            rep(w1), rep(feat1_b), rep(w2), rep(feat2_b),
            rep(w3), rep(dec1_b), rep(w4), rep(dec2_b),
        ],
        out_specs=pl.BlockSpec((BP, 2, NP), lambda i: (i, 0, 0)),
        compiler_params=pltpu.CompilerParams(
            dimension_semantics=("parallel",),
        ),
    )(x, x, cmask, kmask, sel,
      w1, feat1_b, w2, feat2_b, w3, dec1_b, w4, dec2_b)

    return flow.reshape(B, 2, H, W)


# speedup vs baseline: 1.0637x; 1.0031x over previous
import functools

import jax
import jax.numpy as jnp
from jax.experimental import pallas as pl
from jax.experimental.pallas import tpu as pltpu


_F = 16
_HID = 32
_D = 3
_ND = (2 * _D + 1) ** 2
_PAD = 128
_DEC_CH = 80


def _leaky(v):
    return jnp.where(v > 0, v, 0.1 * v)


def _shifts(dmax, W):
    return [dy * W + dx
            for dy in range(-dmax, dmax + 1)
            for dx in range(-dmax, dmax + 1)]


def _pad_lanes(x):
    z = jnp.zeros((x.shape[0], _PAD), x.dtype)
    return jnp.concatenate([z, x, z], axis=-1)


def _conv3x3_bf16(xb, w_ref, b_ref, cmask_ref, W):
    cin, L = xb.shape
    xp = _pad_lanes(xb)
    cols = []
    k = 0
    for dy in (-1, 0, 1):
        for dx in (-1, 0, 1):
            s = dy * W + dx
            sh = xp[:, _PAD + s:_PAD + s + L]
            m = cmask_ref[k]
            if cin <= _F:
                mc = m[:cin]
            else:
                mc = jnp.concatenate([m] * (cin // _F), axis=0)
            cols.append(mc * sh)
            k += 1
    patches = jnp.concatenate(cols, axis=0)
    return jnp.dot(w_ref[...], patches,
                   preferred_element_type=jnp.float32) + b_ref[...]


def _fused_kernel(x1_ref, x2_ref,
                  cmask_ref,
                  kmask_ref,
                  sel_ref,
                  w1_ref, b1_ref, w2_ref, b2_ref,
                  w3_ref, b3_ref, w4_ref, b4_ref,
                  o_ref,
                  pbst,
                  *, H, W, BP):
    NP = H * W
    L = BP * NP

    def extract(x):
        h1 = _leaky(_conv3x3_bf16(x, w1_ref, b1_ref, cmask_ref, W)
                    .astype(jnp.bfloat16))
        h2 = _leaky(_conv3x3_bf16(h1, w2_ref, b2_ref, cmask_ref, W)
                    .astype(jnp.bfloat16))
        return h2

    f1b = extract(x1_ref[0])
    f2b = extract(x2_ref[0])

    f2p = _pad_lanes(f2b)
    for k, s in enumerate(_shifts(_D, W)):
        pbst[k * _F:(k + 1) * _F, :] = f1b * f2p[:, _PAD + s:_PAD + s + L]
    corr_raw = jnp.dot(sel_ref[...], pbst[...],
                       preferred_element_type=jnp.float32)
    corr = _leaky(kmask_ref[...] * corr_raw.astype(jnp.bfloat16))

    zpad = jnp.zeros((_DEC_CH - _F - _ND, L), jnp.bfloat16)
    dec_in = jnp.concatenate([f1b, corr, zpad], axis=0)
    h3 = _leaky(_conv3x3_bf16(dec_in, w3_ref, b3_ref, cmask_ref, W)
                .astype(jnp.bfloat16))
    flow = _conv3x3_bf16(h3, w4_ref, b4_ref, cmask_ref, W)

    for bl in range(BP):
        o_ref[bl] = flow[:, bl * NP:(bl + 1) * NP]


def _im2col_w(w):
    cout = w.shape[3]
    return jnp.transpose(w, (3, 0, 1, 2)).reshape(cout, -1).astype(jnp.bfloat16)


def _masks(L, NP, H, W, dmax, dtype):
    p = jnp.arange(L, dtype=jnp.int32) % NP
    hh = p // W
    ww = p - hh * W
    rows = []
    for dy in range(-dmax, dmax + 1):
        for dx in range(-dmax, dmax + 1):
            m = ((hh + dy >= 0) & (hh + dy < H)
                 & (ww + dx >= 0) & (ww + dx < W))
            rows.append(m)
    return jnp.stack(rows).astype(dtype)


def kernel(data, feat1_w, feat1_b, feat2_w, feat2_b,
           dec1_w, dec1_b, dec2_w, dec2_b):
    B, C, H, W = data.shape
    assert C == 6
    NP = H * W
    BP = next(bp for bp in (8, 4, 2, 1) if B % bp == 0)
    L = BP * NP

    x = (data.astype(jnp.bfloat16)
         .reshape(B, 2, 3, NP).transpose(1, 2, 0, 3).reshape(2, 3, B * NP))

    w1 = _im2col_w(feat1_w)
    w2 = _im2col_w(feat2_w)
    w4 = _im2col_w(dec2_w)
    w3o = _im2col_w(dec1_w).reshape(_HID, 9, _ND + _F)
    w3 = jnp.concatenate(
        [w3o[:, :, _ND:], w3o[:, :, :_ND],
         jnp.zeros((_HID, 9, _DEC_CH - _ND - _F), jnp.bfloat16)],
        axis=2).reshape(_HID, 9 * _DEC_CH)

    cmask = jnp.broadcast_to(
        _masks(L, NP, H, W, 1, jnp.bfloat16)[:, None, :],
        (9, _F, L)).copy()
    kmask = (_masks(L, NP, H, W, _D, jnp.float32) / _F
             ).astype(jnp.bfloat16)
    sel = jnp.kron(jnp.eye(_ND, dtype=jnp.bfloat16),
                   jnp.ones((1, _F), jnp.bfloat16))

    def rep(arr):
        nd = arr.ndim
        return pl.BlockSpec(arr.shape, lambda i, _nd=nd: (0,) * _nd)

    kernel_fn = functools.partial(_fused_kernel, H=H, W=W, BP=BP)

    flow = pl.pallas_call(
        kernel_fn,
        out_shape=jax.ShapeDtypeStruct((B, 2, NP), jnp.float32),
        grid=(B // BP,),
        in_specs=[
            pl.BlockSpec((1, 3, L), lambda i: (0, 0, i)),
            pl.BlockSpec((1, 3, L), lambda i: (1, 0, i)),
            rep(cmask), rep(kmask), rep(sel),
            rep(w1), rep(feat1_b), rep(w2), rep(feat2_b),
            rep(w3), rep(dec1_b), rep(w4), rep(dec2_b),
        ],
        out_specs=pl.BlockSpec((BP, 2, NP), lambda i: (i, 0, 0)),
        scratch_shapes=[pltpu.VMEM((_ND * _F, L), jnp.bfloat16)],
        compiler_params=pltpu.CompilerParams(
            dimension_semantics=("parallel",),
        ),
    )(x, x, cmask, kmask, sel,
      w1, feat1_b, w2, feat2_b, w3, dec1_b, w4, dec2_b)

    return flow.reshape(B, 2, H, W)


# speedup vs baseline: 1.0639x; 1.0002x over previous
import functools

import jax
import jax.numpy as jnp
from jax.experimental import pallas as pl
from jax.experimental.pallas import tpu as pltpu


_F = 16
_HID = 32
_D = 3
_ND = (2 * _D + 1) ** 2
_PAD = 128
_DEC_CH = 80


def _leaky(v):
    return jnp.where(v > 0, v, 0.1 * v)


def _shifts(dmax, W):
    return [dy * W + dx
            for dy in range(-dmax, dmax + 1)
            for dx in range(-dmax, dmax + 1)]


def _pad_lanes(x):
    z = jnp.zeros((x.shape[0], _PAD), x.dtype)
    return jnp.concatenate([z, x, z], axis=-1)


def _conv3x3_bf16(xb, w_ref, b_ref, cmask_ref, W, pbuf=None):
    cin, L = xb.shape
    xp = _pad_lanes(xb)
    cols = []
    k = 0
    for dy in (-1, 0, 1):
        for dx in (-1, 0, 1):
            s = dy * W + dx
            sh = xp[:, _PAD + s:_PAD + s + L]
            m = cmask_ref[k]
            if cin <= _F:
                mc = m[:cin]
            else:
                mc = jnp.concatenate([m] * (cin // _F), axis=0)
            if pbuf is None:
                cols.append(mc * sh)
            else:
                pbuf[k * cin:(k + 1) * cin, :] = mc * sh
            k += 1
    if pbuf is None:
        patches = jnp.concatenate(cols, axis=0)
    else:
        patches = pbuf[...]
    return jnp.dot(w_ref[...], patches,
                   preferred_element_type=jnp.float32) + b_ref[...]


def _fused_kernel(x1_ref, x2_ref,
                  cmask_ref,
                  kmask_ref,
                  sel_ref,
                  w1_ref, b1_ref, w2_ref, b2_ref,
                  w3_ref, b3_ref, w4_ref, b4_ref,
                  o_ref,
                  pbst,
                  pb3,
                  *, H, W, BP):
    NP = H * W
    L = BP * NP

    def extract(x):
        h1 = _leaky(_conv3x3_bf16(x, w1_ref, b1_ref, cmask_ref, W)
                    .astype(jnp.bfloat16))
        h2 = _leaky(_conv3x3_bf16(h1, w2_ref, b2_ref, cmask_ref, W)
                    .astype(jnp.bfloat16))
        return h2

    f1b = extract(x1_ref[0])
    f2b = extract(x2_ref[0])

    f2p = _pad_lanes(f2b)
    for k, s in enumerate(_shifts(_D, W)):
        pbst[k * _F:(k + 1) * _F, :] = f1b * f2p[:, _PAD + s:_PAD + s + L]
    corr_raw = jnp.dot(sel_ref[...], pbst[...],
                       preferred_element_type=jnp.float32)
    corr = _leaky(kmask_ref[...] * corr_raw.astype(jnp.bfloat16))

    zpad = jnp.zeros((_DEC_CH - _F - _ND, L), jnp.bfloat16)
    dec_in = jnp.concatenate([f1b, corr, zpad], axis=0)
    h3 = _leaky(_conv3x3_bf16(dec_in, w3_ref, b3_ref, cmask_ref, W, pb3)
                .astype(jnp.bfloat16))
    flow = _conv3x3_bf16(h3, w4_ref, b4_ref, cmask_ref, W)

    for bl in range(BP):
        o_ref[bl] = flow[:, bl * NP:(bl + 1) * NP]


def _im2col_w(w):
    cout = w.shape[3]
    return jnp.transpose(w, (3, 0, 1, 2)).reshape(cout, -1).astype(jnp.bfloat16)


def _masks(L, NP, H, W, dmax, dtype):
    p = jnp.arange(L, dtype=jnp.int32) % NP
    hh = p // W
    ww = p - hh * W
    rows = []
    for dy in range(-dmax, dmax + 1):
        for dx in range(-dmax, dmax + 1):
            m = ((hh + dy >= 0) & (hh + dy < H)
                 & (ww + dx >= 0) & (ww + dx < W))
            rows.append(m)
    return jnp.stack(rows).astype(dtype)


def kernel(data, feat1_w, feat1_b, feat2_w, feat2_b,
           dec1_w, dec1_b, dec2_w, dec2_b):
    B, C, H, W = data.shape
    assert C == 6
    NP = H * W
    BP = next(bp for bp in (8, 4, 2, 1) if B % bp == 0)
    L = BP * NP

    x = (data.astype(jnp.bfloat16)
         .reshape(B, 2, 3, NP).transpose(1, 2, 0, 3).reshape(2, 3, B * NP))

    w1 = _im2col_w(feat1_w)
    w2 = _im2col_w(feat2_w)
    w4 = _im2col_w(dec2_w)
    w3o = _im2col_w(dec1_w).reshape(_HID, 9, _ND + _F)
    w3 = jnp.concatenate(
        [w3o[:, :, _ND:], w3o[:, :, :_ND],
         jnp.zeros((_HID, 9, _DEC_CH - _ND - _F), jnp.bfloat16)],
        axis=2).reshape(_HID, 9 * _DEC_CH)

    cmask = jnp.broadcast_to(
        _masks(L, NP, H, W, 1, jnp.bfloat16)[:, None, :],
        (9, _F, L)).copy()
    kmask = (_masks(L, NP, H, W, _D, jnp.float32) / _F
             ).astype(jnp.bfloat16)
    sel = jnp.kron(jnp.eye(_ND, dtype=jnp.bfloat16),
                   jnp.ones((1, _F), jnp.bfloat16))

    def rep(arr):
        nd = arr.ndim
        return pl.BlockSpec(arr.shape, lambda i, _nd=nd: (0,) * _nd)

    kernel_fn = functools.partial(_fused_kernel, H=H, W=W, BP=BP)

    flow = pl.pallas_call(
        kernel_fn,
        out_shape=jax.ShapeDtypeStruct((B, 2, NP), jnp.float32),
        grid=(B // BP,),
        in_specs=[
            pl.BlockSpec((1, 3, L), lambda i: (0, 0, i)),
            pl.BlockSpec((1, 3, L), lambda i: (1, 0, i)),
            rep(cmask), rep(kmask), rep(sel),
            rep(w1), rep(feat1_b), rep(w2), rep(feat2_b),
            rep(w3), rep(dec1_b), rep(w4), rep(dec2_b),
        ],
        out_specs=pl.BlockSpec((BP, 2, NP), lambda i: (i, 0, 0)),
        scratch_shapes=[pltpu.VMEM((_ND * _F, L), jnp.bfloat16),
                        pltpu.VMEM((9 * _DEC_CH, L), jnp.bfloat16)],
        compiler_params=pltpu.CompilerParams(
            dimension_semantics=("parallel",),
        ),
    )(x, x, cmask, kmask, sel,
      w1, feat1_b, w2, feat2_b, w3, dec1_b, w4, dec2_b)

    return flow.reshape(B, 2, H, W)


# speedup vs baseline: 1.0653x; 1.0013x over previous
import functools

import jax
import jax.numpy as jnp
from jax.experimental import pallas as pl
from jax.experimental.pallas import tpu as pltpu


_F = 16
_HID = 32
_D = 3
_ND = (2 * _D + 1) ** 2
_PAD = 128
_DEC_CH = 80


def _leaky(v):
    return jnp.where(v > 0, v, 0.1 * v)


def _shifts(dmax, W):
    return [dy * W + dx
            for dy in range(-dmax, dmax + 1)
            for dx in range(-dmax, dmax + 1)]


def _pad_lanes(x):
    z = jnp.zeros((x.shape[0], _PAD), x.dtype)
    return jnp.concatenate([z, x, z], axis=-1)


def _conv3x3_bf16(xb, w_ref, b_ref, cmask_ref, W, pbuf=None):
    cin, L = xb.shape
    xp = _pad_lanes(xb)
    cols = []
    k = 0
    for dy in (-1, 0, 1):
        for dx in (-1, 0, 1):
            s = dy * W + dx
            sh = xp[:, _PAD + s:_PAD + s + L]
            m = cmask_ref[k]
            if cin <= _F:
                mc = m[:cin]
            else:
                mc = jnp.concatenate([m] * (cin // _F), axis=0)
            if pbuf is None:
                cols.append(mc * sh)
            else:
                pbuf[k * cin:(k + 1) * cin, :] = mc * sh
            k += 1
    if pbuf is None:
        patches = jnp.concatenate(cols, axis=0)
    else:
        patches = pbuf[...]
    return jnp.dot(w_ref[...], patches,
                   preferred_element_type=jnp.float32) + b_ref[...]


def _fused_kernel(x1_ref, x2_ref,
                  cmask_ref,
                  kmask_ref,
                  sel_ref,
                  w1_ref, b1_ref, w2_ref, b2_ref,
                  w3_ref, b3_ref, w4_ref, b4_ref,
                  o_ref,
                  pbst,
                  pb3,
                  *, H, W, BP):
    NP = H * W
    L = BP * NP

    def extract(x):
        h1 = _leaky(_conv3x3_bf16(x, w1_ref, b1_ref, cmask_ref, W)
                    .astype(jnp.bfloat16))
        h2 = _leaky(_conv3x3_bf16(h1, w2_ref, b2_ref, cmask_ref, W)
                    .astype(jnp.bfloat16))
        return h2

    f1b = extract(x1_ref[0])
    f2b = extract(x2_ref[0])

    f2p = _pad_lanes(f2b)
    P = _PAD
    for dx in range(-_D, _D + 1):
        u1 = f2p[:, P - 96 + dx:P - 96 + dx + L + 128]
        u2 = f2p[:, P - 64 + dx:P - 64 + dx + L + 128]
        u3 = f2p[:, P - 32 + dx:P - 32 + dx + L + 128]
        views = {-3: u1[:, :L], -2: u2[:, :L], -1: u3[:, :L],
                 0: f2p[:, P + dx:P + dx + L],
                 1: u1[:, 128:], 2: u2[:, 128:], 3: u3[:, 128:]}
        for dy in range(-_D, _D + 1):
            k = (dy + _D) * (2 * _D + 1) + (dx + _D)
            pbst[k * _F:(k + 1) * _F, :] = f1b * views[dy]
    corr_raw = jnp.dot(sel_ref[...], pbst[...],
                       preferred_element_type=jnp.float32)
    corr = _leaky(kmask_ref[...] * corr_raw.astype(jnp.bfloat16))

    zpad = jnp.zeros((_DEC_CH - _F - _ND, L), jnp.bfloat16)
    dec_in = jnp.concatenate([f1b, corr, zpad], axis=0)
    h3 = _leaky(_conv3x3_bf16(dec_in, w3_ref, b3_ref, cmask_ref, W, pb3)
                .astype(jnp.bfloat16))
    flow = _conv3x3_bf16(h3, w4_ref, b4_ref, cmask_ref, W)

    for bl in range(BP):
        o_ref[bl] = flow[:, bl * NP:(bl + 1) * NP]


def _im2col_w(w):
    cout = w.shape[3]
    return jnp.transpose(w, (3, 0, 1, 2)).reshape(cout, -1).astype(jnp.bfloat16)


def _masks(L, NP, H, W, dmax, dtype):
    p = jnp.arange(L, dtype=jnp.int32) % NP
    hh = p // W
    ww = p - hh * W
    rows = []
    for dy in range(-dmax, dmax + 1):
        for dx in range(-dmax, dmax + 1):
            m = ((hh + dy >= 0) & (hh + dy < H)
                 & (ww + dx >= 0) & (ww + dx < W))
            rows.append(m)
    return jnp.stack(rows).astype(dtype)


def kernel(data, feat1_w, feat1_b, feat2_w, feat2_b,
           dec1_w, dec1_b, dec2_w, dec2_b):
    B, C, H, W = data.shape
    assert C == 6
    NP = H * W
    BP = next(bp for bp in (8, 4, 2, 1) if B % bp == 0)
    L = BP * NP

    x = (data.astype(jnp.bfloat16)
         .reshape(B, 2, 3, NP).transpose(1, 2, 0, 3).reshape(2, 3, B * NP))

    w1 = _im2col_w(feat1_w)
    w2 = _im2col_w(feat2_w)
    w4 = _im2col_w(dec2_w)
    w3o = _im2col_w(dec1_w).reshape(_HID, 9, _ND + _F)
    w3 = jnp.concatenate(
        [w3o[:, :, _ND:], w3o[:, :, :_ND],
         jnp.zeros((_HID, 9, _DEC_CH - _ND - _F), jnp.bfloat16)],
        axis=2).reshape(_HID, 9 * _DEC_CH)

    cmask = jnp.broadcast_to(
        _masks(L, NP, H, W, 1, jnp.bfloat16)[:, None, :],
        (9, _F, L)).copy()
    kmask = (_masks(L, NP, H, W, _D, jnp.float32) / _F
             ).astype(jnp.bfloat16)
    sel = jnp.kron(jnp.eye(_ND, dtype=jnp.bfloat16),
                   jnp.ones((1, _F), jnp.bfloat16))

    def rep(arr):
        nd = arr.ndim
        return pl.BlockSpec(arr.shape, lambda i, _nd=nd: (0,) * _nd)

    kernel_fn = functools.partial(_fused_kernel, H=H, W=W, BP=BP)

    flow = pl.pallas_call(
        kernel_fn,
        out_shape=jax.ShapeDtypeStruct((B, 2, NP), jnp.float32),
        grid=(B // BP,),
        in_specs=[
            pl.BlockSpec((1, 3, L), lambda i: (0, 0, i)),
            pl.BlockSpec((1, 3, L), lambda i: (1, 0, i)),
            rep(cmask), rep(kmask), rep(sel),
            rep(w1), rep(feat1_b), rep(w2), rep(feat2_b),
            rep(w3), rep(dec1_b), rep(w4), rep(dec2_b),
        ],
        out_specs=pl.BlockSpec((BP, 2, NP), lambda i: (i, 0, 0)),
        scratch_shapes=[pltpu.VMEM((_ND * _F, L), jnp.bfloat16),
                        pltpu.VMEM((9 * _DEC_CH, L), jnp.bfloat16)],
        compiler_params=pltpu.CompilerParams(
            dimension_semantics=("parallel",),
        ),
    )(x, x, cmask, kmask, sel,
      w1, feat1_b, w2, feat2_b, w3, dec1_b, w4, dec2_b)

    return flow.reshape(B, 2, H, W)


# speedup vs baseline: 1.0653x; 1.0000x over previous
import functools

import jax
import jax.numpy as jnp
from jax.experimental import pallas as pl
from jax.experimental.pallas import tpu as pltpu


_F = 16
_HID = 32
_D = 3
_ND = (2 * _D + 1) ** 2
_PAD = 128
_DEC_CH = 80


def _leaky(v):
    return jnp.where(v > 0, v, 0.1 * v)


def _shifts(dmax, W):
    return [dy * W + dx
            for dy in range(-dmax, dmax + 1)
            for dx in range(-dmax, dmax + 1)]


def _pad_lanes(x):
    z = jnp.zeros((x.shape[0], _PAD), x.dtype)
    return jnp.concatenate([z, x, z], axis=-1)


def _conv3x3_bf16(xb, w_ref, b_ref, cmask_ref, W, pbuf=None):
    cin, L = xb.shape
    xp = _pad_lanes(xb)
    cols = []
    k = 0
    for dy in (-1, 0, 1):
        for dx in (-1, 0, 1):
            s = dy * W + dx
            sh = xp[:, _PAD + s:_PAD + s + L]
            m = cmask_ref[k]
            if cin <= _F:
                mc = m[:cin]
            else:
                mc = jnp.concatenate([m] * (cin // _F), axis=0)
            if pbuf is None:
                cols.append(mc * sh)
            else:
                pbuf[k * cin:(k + 1) * cin, :] = mc * sh
            k += 1
    if pbuf is None:
        patches = jnp.concatenate(cols, axis=0)
    else:
        patches = pbuf[...]
    return jnp.dot(w_ref[...], patches,
                   preferred_element_type=jnp.float32) + b_ref[...]


def _fused_kernel(x1_ref, x2_ref,
                  cmask_ref,
                  kmask_ref,
                  sel_ref,
                  w1_ref, b1_ref, w2_ref, b2_ref,
                  w3_ref, b3_ref, w4_ref, b4_ref,
                  o_ref,
                  pbst,
                  pb3,
                  *, H, W, BP):
    NP = H * W
    L = BP * NP

    def extract(x):
        h1 = _leaky(_conv3x3_bf16(x, w1_ref, b1_ref, cmask_ref, W)
                    .astype(jnp.bfloat16))
        h2 = _leaky(_conv3x3_bf16(h1, w2_ref, b2_ref, cmask_ref, W)
                    .astype(jnp.bfloat16))
        return h2

    f1b = extract(x1_ref[0])
    f2b = extract(x2_ref[0])

    f2p = _pad_lanes(f2b)
    P = _PAD
    step = 128 // W if 128 % W == 0 else None

    def put(dy, dx, view):
        k = (dy + _D) * (2 * _D + 1) + (dx + _D)
        pbst[k * _F:(k + 1) * _F, :] = f1b * view

    for dx in range(-_D, _D + 1):
        done = set()
        for dy in range(-_D, _D + 1):
            if dy in done:
                continue
            s = dy * W + dx
            if step is not None and dy + step <= _D:
                u = f2p[:, P + s:P + s + L + 128]
                put(dy, dx, u[:, :L])
                put(dy + step, dx, u[:, 128:])
                done.add(dy + step)
            else:
                put(dy, dx, f2p[:, P + s:P + s + L])
    corr_raw = jnp.dot(sel_ref[...], pbst[...],
                       preferred_element_type=jnp.float32)
    corr = _leaky(kmask_ref[...] * corr_raw.astype(jnp.bfloat16))

    zpad = jnp.zeros((_DEC_CH - _F - _ND, L), jnp.bfloat16)
    dec_in = jnp.concatenate([f1b, corr, zpad], axis=0)
    h3 = _leaky(_conv3x3_bf16(dec_in, w3_ref, b3_ref, cmask_ref, W, pb3)
                .astype(jnp.bfloat16))
    flow = _conv3x3_bf16(h3, w4_ref, b4_ref, cmask_ref, W)

    for bl in range(BP):
        o_ref[bl] = flow[:, bl * NP:(bl + 1) * NP]


def _im2col_w(w):
    cout = w.shape[3]
    return jnp.transpose(w, (3, 0, 1, 2)).reshape(cout, -1).astype(jnp.bfloat16)


def _masks(L, NP, H, W, dmax, dtype):
    p = jnp.arange(L, dtype=jnp.int32) % NP
    hh = p // W
    ww = p - hh * W
    rows = []
    for dy in range(-dmax, dmax + 1):
        for dx in range(-dmax, dmax + 1):
            m = ((hh + dy >= 0) & (hh + dy < H)
                 & (ww + dx >= 0) & (ww + dx < W))
            rows.append(m)
    return jnp.stack(rows).astype(dtype)


def kernel(data, feat1_w, feat1_b, feat2_w, feat2_b,
           dec1_w, dec1_b, dec2_w, dec2_b):
    B, C, H, W = data.shape
    assert C == 6
    NP = H * W
    BP = next(bp for bp in (8, 4, 2, 1) if B % bp == 0)
    L = BP * NP

    x = (data.astype(jnp.bfloat16)
         .reshape(B, 2, 3, NP).transpose(1, 2, 0, 3).reshape(2, 3, B * NP))

    w1 = _im2col_w(feat1_w)
    w2 = _im2col_w(feat2_w)
    w4 = _im2col_w(dec2_w)
    w3o = _im2col_w(dec1_w).reshape(_HID, 9, _ND + _F)
    w3 = jnp.concatenate(
        [w3o[:, :, _ND:], w3o[:, :, :_ND],
         jnp.zeros((_HID, 9, _DEC_CH - _ND - _F), jnp.bfloat16)],
        axis=2).reshape(_HID, 9 * _DEC_CH)

    cmask = jnp.broadcast_to(
        _masks(L, NP, H, W, 1, jnp.bfloat16)[:, None, :],
        (9, _F, L)).copy()
    kmask = (_masks(L, NP, H, W, _D, jnp.float32) / _F
             ).astype(jnp.bfloat16)
    sel = jnp.kron(jnp.eye(_ND, dtype=jnp.bfloat16),
                   jnp.ones((1, _F), jnp.bfloat16))

    def rep(arr):
        nd = arr.ndim
        return pl.BlockSpec(arr.shape, lambda i, _nd=nd: (0,) * _nd)

    kernel_fn = functools.partial(_fused_kernel, H=H, W=W, BP=BP)

    flow = pl.pallas_call(
        kernel_fn,
        out_shape=jax.ShapeDtypeStruct((B, 2, NP), jnp.float32),
        grid=(B // BP,),
        in_specs=[
            pl.BlockSpec((1, 3, L), lambda i: (0, 0, i)),
            pl.BlockSpec((1, 3, L), lambda i: (1, 0, i)),
            rep(cmask), rep(kmask), rep(sel),
            rep(w1), rep(feat1_b), rep(w2), rep(feat2_b),
            rep(w3), rep(dec1_b), rep(w4), rep(dec2_b),
        ],
        out_specs=pl.BlockSpec((BP, 2, NP), lambda i: (i, 0, 0)),
        scratch_shapes=[pltpu.VMEM((_ND * _F, L), jnp.bfloat16),
                        pltpu.VMEM((9 * _DEC_CH, L), jnp.bfloat16)],
        compiler_params=pltpu.CompilerParams(
            dimension_semantics=("parallel",),
        ),
    )(x, x, cmask, kmask, sel,
      w1, feat1_b, w2, feat2_b, w3, dec1_b, w4, dec2_b)

    return flow.reshape(B, 2, H, W)
